# baseline (speedup 1.0000x reference)
import functools

import jax
import jax.numpy as jnp
from jax import lax
from jax.experimental import pallas as pl
from jax.experimental.pallas import tpu as pltpu
from jax.experimental.pallas import tpu_sc as plsc

_B, _F, _V, _D = 4096, 26, 100000, 16
_R = _F * _D
_NC, _NS = 2, 16
_NW = _NC * _NS
_RPW = _R // _NW
_LANES = 16


def _make_gather_body(fa16, rpw):
    def body(emb_hbm, svt_hbm, embt_out, vrows_v, rows_v, sem):
        wid = lax.axis_index("s") * _NC + lax.axis_index("c")
        r0 = wid * rpw
        f0 = (fa16 + r0) // _D
        pltpu.sync_copy(svt_hbm.at[f0], vrows_v.at[0])
        f1 = (fa16 + r0 + rpw - 1) // _D
        pltpu.sync_copy(svt_hbm.at[f1], vrows_v.at[1])

        def fire(k, _):
            r = r0 + k
            sel = (fa16 + r) // _D - f0
            pltpu.make_async_copy(
                emb_hbm.at[pl.ds(r * _V, _V)].at[vrows_v.at[sel]],
                rows_v.at[k], sem).start()
            return 0

        lax.fori_loop(0, rpw, fire, 0)

        def drain(k, _):
            r = r0 + k
            sel = (fa16 + r) // _D - f0
            pltpu.make_async_copy(
                emb_hbm.at[pl.ds(r * _V, _V)].at[vrows_v.at[sel]],
                rows_v.at[k], sem).wait()
            return 0

        lax.fori_loop(0, rpw, drain, 0)
        pltpu.sync_copy(rows_v, embt_out.at[pl.ds(r0, rpw)])

    return body


def _lin_gather_body(lin_hbm, svt_hbm, lint_out, vrow_v, lrow_v, sem):
    wid = lax.axis_index("s") * _NC + lax.axis_index("c")

    @pl.when(wid < _F)
    def _():
        pltpu.sync_copy(svt_hbm.at[wid], vrow_v)
        cp = pltpu.make_async_copy(
            lin_hbm.at[pl.ds(wid * _V, _V)].at[vrow_v], lrow_v, sem)
        cp.start()
        cp.wait()
        pltpu.sync_copy(lrow_v, lint_out.at[wid])


def _sc_gather_chunk(emb_flat_chunk, svt, fa, rpw):
    mesh = plsc.VectorSubcoreMesh(core_axis_name="c", subcore_axis_name="s")
    f = pl.kernel(
        _make_gather_body(fa * _D, rpw),
        out_type=jax.ShapeDtypeStruct((_NW * rpw, _B), jnp.float32),
        mesh=mesh,
        scratch_types=[
            pltpu.VMEM((2, _B), jnp.int32),
            pltpu.VMEM((rpw, _B), jnp.float32),
            pltpu.SemaphoreType.DMA,
        ],
        compiler_params=pltpu.CompilerParams(use_tc_tiling_on_sc=False),
    )
    return f(emb_flat_chunk, svt)


def _sc_lin_gather(lin_flat, svt):
    mesh = plsc.VectorSubcoreMesh(core_axis_name="c", subcore_axis_name="s")
    f = pl.kernel(
        _lin_gather_body,
        out_type=jax.ShapeDtypeStruct((_F, _B), jnp.float32),
        mesh=mesh,
        scratch_types=[
            pltpu.VMEM((_B,), jnp.int32),
            pltpu.VMEM((_B,), jnp.float32),
            pltpu.SemaphoreType.DMA,
        ],
        compiler_params=pltpu.CompilerParams(use_tc_tiling_on_sc=False),
    )
    return f(lin_flat, svt)


def _dense_body(xt_ref, lt_ref, st_ref, w1_ref, b1_ref, w2_ref, b2_ref,
                w3_ref, b3_ref, wo_ref, bo_ref, out_ref):
    x = xt_ref[...]
    t = jnp.dot(st_ref[...], x, preferred_element_type=jnp.float32)
    fm = 0.5 * (jnp.sum(t * t, axis=0, keepdims=True)
                - jnp.sum(x * x, axis=0, keepdims=True))
    linear = jnp.sum(lt_ref[...], axis=0, keepdims=True)
    h = jnp.maximum(
        jnp.dot(w1_ref[...], x, preferred_element_type=jnp.float32)
        + b1_ref[...], 0.0)
    h = jnp.maximum(
        jnp.dot(w2_ref[...], h, preferred_element_type=jnp.float32)
        + b2_ref[...], 0.0)
    h = jnp.maximum(
        jnp.dot(w3_ref[...], h, preferred_element_type=jnp.float32)
        + b3_ref[...], 0.0)
    dnn = jnp.sum(h * wo_ref[...], axis=0, keepdims=True)
    out_ref[...] = jax.nn.sigmoid(linear + fm + dnn + bo_ref[...])


def _tc_dense(xt, lt, st, w1t, b1, w2t, b2, w3t, b3, wo, bo, bn=1024):
    grid = (_B // bn,)
    full = lambda shape: pl.BlockSpec(shape, lambda i: (0, 0))
    return pl.pallas_call(
        _dense_body,
        grid=grid,
        in_specs=[
            pl.BlockSpec((_R, bn), lambda i: (0, i)),
            pl.BlockSpec((_F, bn), lambda i: (0, i)),
            full((_D, _R)),
            full((256, _R)),
            full((256, 1)),
            full((128, 256)),
            full((128, 1)),
            full((64, 128)),
            full((64, 1)),
            full((64, 1)),
            full((1, 1)),
        ],
        out_specs=pl.BlockSpec((1, bn), lambda i: (0, i)),
        out_shape=jax.ShapeDtypeStruct((1, _B), jnp.float32),
    )(xt, lt, st, w1t, b1, w2t, b2, w3t, b3, wo, bo)


def kernel(sparse_features, embed_tables, linear_tables,
           W1, b1, g1, be1, W2, b2, g2, be2, W3, b3, g3, be3, Wo, bo):
    lin_flat = linear_tables.reshape(_F * _V)
    svt = jnp.transpose(sparse_features).astype(jnp.int32)

    chunks = []
    for fa, fb in ((0, 6), (6, 12), (12, 18), (18, 26)):
        nf = fb - fa
        emb_c = jnp.transpose(embed_tables[fa:fb], (0, 2, 1)).reshape(
            nf * _D * _V)
        chunks.append(_sc_gather_chunk(emb_c, svt, fa, nf * _D // _NW))
    xt = jnp.concatenate(chunks, axis=0)
    lt = _sc_lin_gather(lin_flat, svt)

    st = jnp.tile(jnp.eye(_D, dtype=jnp.float32), (1, _F))
    inv = 1.0 / jnp.sqrt(jnp.float32(1.0 + 1e-5))
    s1, s2, s3 = g1 * inv, g2 * inv, g3 * inv
    w1t = (W1 * s1[None, :]).T
    b1f = (b1 * s1 + be1)[:, None]
    w2t = (W2 * s2[None, :]).T
    b2f = (b2 * s2 + be2)[:, None]
    w3t = (W3 * s3[None, :]).T
    b3f = (b3 * s3 + be3)[:, None]

    out = _tc_dense(xt, lt, st, w1t, b1f, w2t, b2f, w3t, b3f,
                    Wo, bo.reshape(1, 1))
    return out.reshape(_B)


# speedup vs baseline: 1.3133x; 1.3133x over previous
import functools

import jax
import jax.numpy as jnp
from jax import lax
from jax.experimental import pallas as pl
from jax.experimental.pallas import tpu as pltpu
from jax.experimental.pallas import tpu_sc as plsc

_B, _F, _V, _D = 4096, 26, 100000, 16
_R = _F * _D
_NC, _NS = 2, 16
_NW = _NC * _NS
_RPW = _R // _NW
_LANES = 16


def _make_gather_body(fa16, rpw):
    def body(emb_hbm, svt_hbm, embt_out, vrows_v, rows_v, sem):
        wid = lax.axis_index("s") * _NC + lax.axis_index("c")
        r0 = wid * rpw
        f0 = (fa16 + r0) // _D
        pltpu.sync_copy(svt_hbm.at[f0], vrows_v.at[0])
        f1 = (fa16 + r0 + rpw - 1) // _D
        pltpu.sync_copy(svt_hbm.at[f1], vrows_v.at[1])

        def fire(k, _):
            r = r0 + k
            sel = (fa16 + r) // _D - f0
            pltpu.make_async_copy(
                emb_hbm.at[pl.ds(r * _V, _V)].at[vrows_v.at[sel]],
                rows_v.at[k], sem).start()
            return 0

        lax.fori_loop(0, rpw, fire, 0)

        def drain(k, _):
            r = r0 + k
            sel = (fa16 + r) // _D - f0
            pltpu.make_async_copy(
                emb_hbm.at[pl.ds(r * _V, _V)].at[vrows_v.at[sel]],
                rows_v.at[k], sem).wait()
            return 0

        lax.fori_loop(0, rpw, drain, 0)
        pltpu.sync_copy(rows_v, embt_out.at[pl.ds(r0, rpw)])

    return body


def _lin_gather_body(lin_hbm, svt_hbm, lint_out, vrow_v, lrow_v, sem):
    wid = lax.axis_index("s") * _NC + lax.axis_index("c")

    @pl.when(wid < _F)
    def _():
        pltpu.sync_copy(svt_hbm.at[wid], vrow_v)
        cp = pltpu.make_async_copy(
            lin_hbm.at[pl.ds(wid * _V, _V)].at[vrow_v], lrow_v, sem)
        cp.start()
        cp.wait()
        pltpu.sync_copy(lrow_v, lint_out.at[wid])


def _sc_gather_chunk(emb_flat_chunk, svt, fa, rpw):
    mesh = plsc.VectorSubcoreMesh(core_axis_name="c", subcore_axis_name="s")
    f = pl.kernel(
        _make_gather_body(fa * _D, rpw),
        out_type=jax.ShapeDtypeStruct((_NW * rpw, _B), jnp.float32),
        mesh=mesh,
        scratch_types=[
            pltpu.VMEM((2, _B), jnp.int32),
            pltpu.VMEM((rpw, _B), jnp.float32),
            pltpu.SemaphoreType.DMA,
        ],
        compiler_params=pltpu.CompilerParams(use_tc_tiling_on_sc=False),
    )
    return f(emb_flat_chunk, svt)


def _sc_lin_gather(lin_flat, svt):
    mesh = plsc.VectorSubcoreMesh(core_axis_name="c", subcore_axis_name="s")
    f = pl.kernel(
        _lin_gather_body,
        out_type=jax.ShapeDtypeStruct((_F, _B), jnp.float32),
        mesh=mesh,
        scratch_types=[
            pltpu.VMEM((_B,), jnp.int32),
            pltpu.VMEM((_B,), jnp.float32),
            pltpu.SemaphoreType.DMA,
        ],
        compiler_params=pltpu.CompilerParams(use_tc_tiling_on_sc=False),
    )
    return f(lin_flat, svt)


def _dense_body(xt_ref, lt_ref, st_ref, w1_ref, b1_ref, w2_ref, b2_ref,
                w3_ref, b3_ref, wo_ref, bo_ref, out_ref):
    x = xt_ref[...]
    t = jnp.dot(st_ref[...], x, preferred_element_type=jnp.float32)
    fm = 0.5 * (jnp.sum(t * t, axis=0, keepdims=True)
                - jnp.sum(x * x, axis=0, keepdims=True))
    linear = jnp.sum(lt_ref[...], axis=0, keepdims=True)
    h = jnp.maximum(
        jnp.dot(w1_ref[...], x, preferred_element_type=jnp.float32)
        + b1_ref[...], 0.0)
    h = jnp.maximum(
        jnp.dot(w2_ref[...], h, preferred_element_type=jnp.float32)
        + b2_ref[...], 0.0)
    h = jnp.maximum(
        jnp.dot(w3_ref[...], h, preferred_element_type=jnp.float32)
        + b3_ref[...], 0.0)
    dnn = jnp.sum(h * wo_ref[...], axis=0, keepdims=True)
    out_ref[...] = jax.nn.sigmoid(linear + fm + dnn + bo_ref[...])


def _tc_dense(xt, lt, st, w1t, b1, w2t, b2, w3t, b3, wo, bo, bn=1024):
    grid = (_B // bn,)
    full = lambda shape: pl.BlockSpec(shape, lambda i: (0, 0))
    return pl.pallas_call(
        _dense_body,
        grid=grid,
        in_specs=[
            pl.BlockSpec((_R, bn), lambda i: (0, i)),
            pl.BlockSpec((_F, bn), lambda i: (0, i)),
            full((_D, _R)),
            full((256, _R)),
            full((256, 1)),
            full((128, 256)),
            full((128, 1)),
            full((64, 128)),
            full((64, 1)),
            full((64, 1)),
            full((1, 1)),
        ],
        out_specs=pl.BlockSpec((1, bn), lambda i: (0, i)),
        out_shape=jax.ShapeDtypeStruct((1, _B), jnp.float32),
    )(xt, lt, st, w1t, b1, w2t, b2, w3t, b3, wo, bo)


def kernel(sparse_features, embed_tables, linear_tables,
           W1, b1, g1, be1, W2, b2, g2, be2, W3, b3, g3, be3, Wo, bo):
    lin_flat = linear_tables.reshape(_F * _V)
    svt = jnp.transpose(sparse_features).astype(jnp.int32)

    emb_flat = jnp.transpose(embed_tables, (0, 2, 1)).reshape(_R * _V)
    xt = _sc_gather_chunk(emb_flat, svt, 0, _RPW)
    lt = _sc_lin_gather(lin_flat, svt)

    st = jnp.tile(jnp.eye(_D, dtype=jnp.float32), (1, _F))
    inv = 1.0 / jnp.sqrt(jnp.float32(1.0 + 1e-5))
    s1, s2, s3 = g1 * inv, g2 * inv, g3 * inv
    w1t = (W1 * s1[None, :]).T
    b1f = (b1 * s1 + be1)[:, None]
    w2t = (W2 * s2[None, :]).T
    b2f = (b2 * s2 + be2)[:, None]
    w3t = (W3 * s3[None, :]).T
    b3f = (b3 * s3 + be3)[:, None]

    out = _tc_dense(xt, lt, st, w1t, b1f, w2t, b2f, w3t, b3f,
                    Wo, bo.reshape(1, 1))
    return out.reshape(_B)
